# Initial kernel scaffold; baseline (speedup 1.0000x reference)
import jax
import jax.numpy as jnp
from jax.experimental import pallas as pl


def kernel(x, edge_index, W1, b1, W2, b2, W3, b3):
    raise NotImplementedError("write your pallas kernel here")


# speedup vs baseline: 16.0368x; 16.0368x over previous
import functools

import jax
import jax.numpy as jnp
from jax import lax
from jax.experimental import pallas as pl
from jax.experimental.pallas import tpu as pltpu
from jax.experimental.pallas import tpu_sc as plsc

N_NODES = 10000
N_EDGES = 320000
NP = 10240

NC = 2
NS = 16
NW = NC * NS
CH = 125
NCHUNKS = N_EDGES // CH
CH_PER_W = NCHUNKS // NW
RPS = NP // NS
ZCH = 128
DEGW = 16

_mesh = plsc.VectorSubcoreMesh(core_axis_name="c", subcore_axis_name="s")
_sc_params = pltpu.CompilerParams(use_tc_tiling_on_sc=False)


def _zero_fill(buf, h):
    @pl.loop(0, buf.shape[0])
    def _(r):
        @pl.loop(0, h // 16)
        def _(c):
            buf[r, pl.ds(c * 16, 16)] = jnp.zeros((16,), jnp.float32)


def _make_edge_scatter(h):
    @functools.partial(
        pl.kernel,
        out_type=jax.ShapeDtypeStruct((NC, NP, h), jnp.float32),
        mesh=_mesh,
        scratch_types=[
            pltpu.VMEM((1, CH), jnp.int32),
            pltpu.VMEM((1, CH), jnp.int32),
            pltpu.VMEM((CH, h), jnp.float32),
            pltpu.VMEM((ZCH, h), jnp.float32),
            pltpu.VMEM_SHARED((NP, h), jnp.float32),
        ],
        compiler_params=_sc_params,
    )
    def k(src_hbm, dst_hbm, y_hbm, out_hbm, sidx, didx, rows, zbuf, acc):
        cid = lax.axis_index("c")
        sid = lax.axis_index("s")
        wid = cid * NS + sid

        _zero_fill(zbuf, h)

        @pl.loop(0, RPS // ZCH)
        def _(t):
            pltpu.sync_copy(zbuf, acc.at[pl.ds(sid * RPS + t * ZCH, ZCH)])

        plsc.subcore_barrier()

        @pl.loop(0, CH_PER_W)
        def _(t):
            j = wid * CH_PER_W + t
            pltpu.sync_copy(src_hbm.at[j], sidx)
            pltpu.sync_copy(dst_hbm.at[j], didx)
            pltpu.sync_copy(y_hbm.at[sidx.at[0]], rows)
            pltpu.sync_copy(rows, acc.at[didx.at[0]], add=True)

        plsc.subcore_barrier()
        pltpu.sync_copy(acc.at[pl.ds(sid * RPS, RPS)],
                        out_hbm.at[cid, pl.ds(sid * RPS, RPS)])

    return k


_edge_scatter = {h: _make_edge_scatter(h) for h in (32, 64)}


@functools.partial(
    pl.kernel,
    out_type=jax.ShapeDtypeStruct((NC, NP, DEGW), jnp.float32),
    mesh=_mesh,
    scratch_types=[
        pltpu.VMEM((1, CH), jnp.int32),
        pltpu.VMEM((CH, DEGW), jnp.float32),
        pltpu.VMEM((ZCH, DEGW), jnp.float32),
        pltpu.VMEM_SHARED((NP, DEGW), jnp.float32),
    ],
    compiler_params=_sc_params,
)
def _deg_kernel(dst_hbm, out_hbm, didx, obuf, zbuf, acc):
    cid = lax.axis_index("c")
    sid = lax.axis_index("s")
    wid = cid * NS + sid

    _zero_fill(zbuf, DEGW)

    @pl.loop(0, obuf.shape[0])
    def _(r):
        obuf[r, pl.ds(0, 16)] = jnp.ones((16,), jnp.float32)

    @pl.loop(0, RPS // ZCH)
    def _(t):
        pltpu.sync_copy(zbuf, acc.at[pl.ds(sid * RPS + t * ZCH, ZCH)])

    plsc.subcore_barrier()

    @pl.loop(0, CH_PER_W)
    def _(t):
        j = wid * CH_PER_W + t
        pltpu.sync_copy(dst_hbm.at[j], didx)
        pltpu.sync_copy(obuf, acc.at[didx.at[0]], add=True)

    plsc.subcore_barrier()
    pltpu.sync_copy(acc.at[pl.ds(sid * RPS, RPS)],
                    out_hbm.at[cid, pl.ds(sid * RPS, RPS)])


def _stage0_body(dega_ref, degb_ref, x_ref, w_ref, dinv_ref, y_ref):
    deg = (dega_ref[pl.ds(0, N_NODES), 0:1]
           + degb_ref[pl.ds(0, N_NODES), 0:1] + 1.0)
    dinv = lax.rsqrt(deg)
    dinv_ref[...] = dinv
    y_ref[...] = dinv * jnp.dot(x_ref[...], w_ref[...],
                                preferred_element_type=jnp.float32)


def _stage_mid_body(acca_ref, accb_ref, y_ref, dinv_ref, b_ref, w_ref, yout_ref):
    dinv = dinv_ref[...]
    acc = acca_ref[pl.ds(0, N_NODES), :] + accb_ref[pl.ds(0, N_NODES), :]
    h = jax.nn.relu(dinv * (acc + y_ref[...]) + b_ref[...])
    yout_ref[...] = dinv * jnp.dot(h, w_ref[...],
                                   preferred_element_type=jnp.float32)


def _stage3_body(acca_ref, accb_ref, y_ref, dinv_ref, b_ref, out_ref):
    acc = acca_ref[pl.ds(0, N_NODES), :] + accb_ref[pl.ds(0, N_NODES), :]
    z = dinv_ref[...] * (acc + y_ref[...]) + b_ref[...]
    m = jnp.max(z, axis=1, keepdims=True)
    e = jnp.exp(z - m)
    out_ref[...] = (z - m) - jnp.log(jnp.sum(e, axis=1, keepdims=True))


def _stage0(dega, degb, x, w):
    return pl.pallas_call(
        _stage0_body,
        out_shape=(jax.ShapeDtypeStruct((N_NODES, 1), jnp.float32),
                   jax.ShapeDtypeStruct((N_NODES, w.shape[1]), jnp.float32)),
    )(dega, degb, x, w)


def _stage_mid(acca, accb, y, dinv, b, w):
    return pl.pallas_call(
        _stage_mid_body,
        out_shape=jax.ShapeDtypeStruct((N_NODES, w.shape[1]), jnp.float32),
    )(acca, accb, y, dinv, b.reshape(1, -1), w)


def _stage3(acca, accb, y, dinv, b):
    return pl.pallas_call(
        _stage3_body,
        out_shape=jax.ShapeDtypeStruct((N_NODES, y.shape[1]), jnp.float32),
    )(acca, accb, y, dinv, b.reshape(1, -1))


def kernel(x, edge_index, W1, b1, W2, b2, W3, b3):
    src = edge_index[0].reshape(NCHUNKS, 1, CH)
    dst = edge_index[1].reshape(NCHUNKS, 1, CH)

    degp = _deg_kernel(dst)
    dinv, y1 = _stage0(degp[0], degp[1], x, W1)

    acc1 = _edge_scatter[32](src, dst, y1)
    y2 = _stage_mid(acc1[0], acc1[1], y1, dinv, b1, W2)

    acc2 = _edge_scatter[64](src, dst, y2)
    y3 = _stage_mid(acc2[0], acc2[1], y2, dinv, b2, W3)

    acc3 = _edge_scatter[64](src, dst, y3)
    return _stage3(acc3[0], acc3[1], y3, dinv, b3)


# speedup vs baseline: 33.4420x; 2.0853x over previous
import functools

import jax
import jax.numpy as jnp
from jax import lax
from jax.experimental import pallas as pl
from jax.experimental.pallas import tpu as pltpu
from jax.experimental.pallas import tpu_sc as plsc

N_NODES = 10000
N_EDGES = 320000
NP = 10240

NC = 2
NS = 16
NW = NC * NS
CH = 125
NCHUNKS = N_EDGES // CH
CH_PER_W = NCHUNKS // NW
RPS = NP // NS
ZCH = 128
DEGW = 16

_mesh = plsc.VectorSubcoreMesh(core_axis_name="c", subcore_axis_name="s")
_sc_params = pltpu.CompilerParams(use_tc_tiling_on_sc=False)


def _zero_fill(buf, h):
    @pl.loop(0, buf.shape[0])
    def _(r):
        @pl.loop(0, h // 16)
        def _(c):
            buf[r, pl.ds(c * 16, 16)] = jnp.zeros((16,), jnp.float32)


def _make_edge_scatter(h):
    @functools.partial(
        pl.kernel,
        out_type=jax.ShapeDtypeStruct((NC, NP, h), jnp.float32),
        mesh=_mesh,
        scratch_types=[
            pltpu.VMEM((CH_PER_W, CH), jnp.int32),
            pltpu.VMEM((CH_PER_W, CH), jnp.int32),
            pltpu.VMEM((CH, h), jnp.float32),
            pltpu.VMEM((CH, h), jnp.float32),
            pltpu.VMEM((ZCH, h), jnp.float32),
            pltpu.VMEM_SHARED((NP, h), jnp.float32),
            pltpu.SemaphoreType.DMA,
            pltpu.SemaphoreType.DMA,
        ],
        compiler_params=_sc_params,
    )
    def k(src_hbm, dst_hbm, y_hbm, out_hbm, sidx, didx, rows0, rows1, zbuf,
          acc, sem0, sem1):
        cid = lax.axis_index("c")
        sid = lax.axis_index("s")
        wid = cid * NS + sid

        _zero_fill(zbuf, h)

        @pl.loop(0, RPS // ZCH)
        def _(t):
            pltpu.sync_copy(zbuf, acc.at[pl.ds(sid * RPS + t * ZCH, ZCH)])

        pltpu.sync_copy(src_hbm.at[pl.ds(wid * CH_PER_W, CH_PER_W)], sidx)
        pltpu.sync_copy(dst_hbm.at[pl.ds(wid * CH_PER_W, CH_PER_W)], didx)
        plsc.subcore_barrier()

        def gather(t, buf, sem):
            pltpu.make_async_copy(y_hbm.at[sidx.at[t]], buf, sem).start()

        def gather_wait(buf, sem):
            pltpu.make_async_copy(y_hbm.at[sidx.at[0]], buf, sem).wait()

        def scat(t, buf):
            pltpu.sync_copy(buf, acc.at[didx.at[t]], add=True)

        gather(0, rows0, sem0)
        gather(1, rows1, sem1)

        @pl.loop(0, CH_PER_W, step=2)
        def _(t):
            gather_wait(rows0, sem0)
            scat(t, rows0)

            @pl.when(t + 2 < CH_PER_W)
            def _():
                gather(t + 2, rows0, sem0)

            gather_wait(rows1, sem1)
            scat(t + 1, rows1)

            @pl.when(t + 3 < CH_PER_W)
            def _():
                gather(t + 3, rows1, sem1)

        plsc.subcore_barrier()
        pltpu.sync_copy(acc.at[pl.ds(sid * RPS, RPS)],
                        out_hbm.at[cid, pl.ds(sid * RPS, RPS)])

    return k


_edge_scatter = {h: _make_edge_scatter(h) for h in (32, 64)}


@functools.partial(
    pl.kernel,
    out_type=jax.ShapeDtypeStruct((NC, NP, DEGW), jnp.float32),
    mesh=_mesh,
    scratch_types=[
        pltpu.VMEM((CH_PER_W, CH), jnp.int32),
        pltpu.VMEM((CH, DEGW), jnp.float32),
        pltpu.VMEM((ZCH, DEGW), jnp.float32),
        pltpu.VMEM_SHARED((NP, DEGW), jnp.float32),
    ],
    compiler_params=_sc_params,
)
def _deg_kernel(dst_hbm, out_hbm, didx, obuf, zbuf, acc):
    cid = lax.axis_index("c")
    sid = lax.axis_index("s")
    wid = cid * NS + sid

    _zero_fill(zbuf, DEGW)

    @pl.loop(0, obuf.shape[0])
    def _(r):
        obuf[r, pl.ds(0, 16)] = jnp.ones((16,), jnp.float32)

    @pl.loop(0, RPS // ZCH)
    def _(t):
        pltpu.sync_copy(zbuf, acc.at[pl.ds(sid * RPS + t * ZCH, ZCH)])

    pltpu.sync_copy(dst_hbm.at[pl.ds(wid * CH_PER_W, CH_PER_W)], didx)
    plsc.subcore_barrier()

    @pl.loop(0, CH_PER_W)
    def _(t):
        pltpu.sync_copy(obuf, acc.at[didx.at[t]], add=True)

    plsc.subcore_barrier()
    pltpu.sync_copy(acc.at[pl.ds(sid * RPS, RPS)],
                    out_hbm.at[cid, pl.ds(sid * RPS, RPS)])


def _stage0_body(dega_ref, degb_ref, x_ref, w_ref, dinv_ref, y_ref):
    deg = (dega_ref[pl.ds(0, N_NODES), 0:1]
           + degb_ref[pl.ds(0, N_NODES), 0:1] + 1.0)
    dinv = lax.rsqrt(deg)
    dinv_ref[...] = dinv
    y_ref[...] = dinv * jnp.dot(x_ref[...], w_ref[...],
                                preferred_element_type=jnp.float32)


def _stage_mid_body(acca_ref, accb_ref, y_ref, dinv_ref, b_ref, w_ref, yout_ref):
    dinv = dinv_ref[...]
    acc = acca_ref[pl.ds(0, N_NODES), :] + accb_ref[pl.ds(0, N_NODES), :]
    h = jax.nn.relu(dinv * (acc + y_ref[...]) + b_ref[...])
    yout_ref[...] = dinv * jnp.dot(h, w_ref[...],
                                   preferred_element_type=jnp.float32)


def _stage3_body(acca_ref, accb_ref, y_ref, dinv_ref, b_ref, out_ref):
    acc = acca_ref[pl.ds(0, N_NODES), :] + accb_ref[pl.ds(0, N_NODES), :]
    z = dinv_ref[...] * (acc + y_ref[...]) + b_ref[...]
    m = jnp.max(z, axis=1, keepdims=True)
    e = jnp.exp(z - m)
    out_ref[...] = (z - m) - jnp.log(jnp.sum(e, axis=1, keepdims=True))


def _stage0(dega, degb, x, w):
    return pl.pallas_call(
        _stage0_body,
        out_shape=(jax.ShapeDtypeStruct((N_NODES, 1), jnp.float32),
                   jax.ShapeDtypeStruct((N_NODES, w.shape[1]), jnp.float32)),
    )(dega, degb, x, w)


def _stage_mid(acca, accb, y, dinv, b, w):
    return pl.pallas_call(
        _stage_mid_body,
        out_shape=jax.ShapeDtypeStruct((N_NODES, w.shape[1]), jnp.float32),
    )(acca, accb, y, dinv, b.reshape(1, -1), w)


def _stage3(acca, accb, y, dinv, b):
    return pl.pallas_call(
        _stage3_body,
        out_shape=jax.ShapeDtypeStruct((N_NODES, y.shape[1]), jnp.float32),
    )(acca, accb, y, dinv, b.reshape(1, -1))


def kernel(x, edge_index, W1, b1, W2, b2, W3, b3):
    src = edge_index[0].reshape(NCHUNKS, CH)
    dst = edge_index[1].reshape(NCHUNKS, CH)

    degp = _deg_kernel(dst)
    dinv, y1 = _stage0(degp[0], degp[1], x, W1)

    acc1 = _edge_scatter[32](src, dst, y1)
    y2 = _stage_mid(acc1[0], acc1[1], y1, dinv, b1, W2)

    acc2 = _edge_scatter[64](src, dst, y2)
    y3 = _stage_mid(acc2[0], acc2[1], y2, dinv, b2, W3)

    acc3 = _edge_scatter[64](src, dst, y3)
    return _stage3(acc3[0], acc3[1], y3, dinv, b3)


# speedup vs baseline: 39.8791x; 1.1925x over previous
import functools

import jax
import jax.numpy as jnp
from jax import lax
from jax.experimental import pallas as pl
from jax.experimental.pallas import tpu as pltpu
from jax.experimental.pallas import tpu_sc as plsc

N_NODES = 10000
N_EDGES = 320000
NP = 10240

NC = 2
NS = 16
NW = NC * NS
CH = 125
NCHUNKS = N_EDGES // CH
CH_PER_W = NCHUNKS // NW
RPS = NP // NS
ZCH = 128
NBUF = 4
DEGW = 16

_mesh = plsc.VectorSubcoreMesh(core_axis_name="c", subcore_axis_name="s")
_sc_params = pltpu.CompilerParams(use_tc_tiling_on_sc=False)


def _zero_fill(buf, h):
    @pl.loop(0, buf.shape[0])
    def _(r):
        @pl.loop(0, h // 16)
        def _(c):
            buf[r, pl.ds(c * 16, 16)] = jnp.zeros((16,), jnp.float32)


def _make_edge_scatter(h):
    @functools.partial(
        pl.kernel,
        out_type=jax.ShapeDtypeStruct((NC, NP, h), jnp.float32),
        mesh=_mesh,
        scratch_types=[
            pltpu.VMEM((CH_PER_W, CH), jnp.int32),
            pltpu.VMEM((CH_PER_W, CH), jnp.int32),
            [pltpu.VMEM((CH, h), jnp.float32) for _ in range(NBUF)],
            pltpu.VMEM((ZCH, h), jnp.float32),
            pltpu.VMEM_SHARED((NP, h), jnp.float32),
            [pltpu.SemaphoreType.DMA for _ in range(NBUF)],
            [pltpu.SemaphoreType.DMA for _ in range(NBUF)],
        ],
        compiler_params=_sc_params,
    )
    def k(src_hbm, dst_hbm, y_hbm, out_hbm, sidx, didx, bufs, zbuf,
          acc, gsems, ssems):
        cid = lax.axis_index("c")
        sid = lax.axis_index("s")
        wid = cid * NS + sid

        _zero_fill(zbuf, h)

        @pl.loop(0, RPS // ZCH)
        def _(t):
            pltpu.sync_copy(zbuf, acc.at[pl.ds(sid * RPS + t * ZCH, ZCH)])

        pltpu.sync_copy(src_hbm.at[pl.ds(wid * CH_PER_W, CH_PER_W)], sidx)
        pltpu.sync_copy(dst_hbm.at[pl.ds(wid * CH_PER_W, CH_PER_W)], didx)
        plsc.subcore_barrier()

        def gather(t, b):
            pltpu.make_async_copy(y_hbm.at[sidx.at[t]], bufs[b], gsems[b]).start()

        def gather_wait(b):
            pltpu.make_async_copy(y_hbm.at[sidx.at[0]], bufs[b], gsems[b]).wait()

        def scat(t, b):
            pltpu.async_copy(bufs[b], acc.at[didx.at[t]], ssems[b], add=True)

        def scat_wait(b):
            pltpu.make_async_copy(bufs[b], acc.at[didx.at[0]], ssems[b]).wait()

        for b in range(NBUF):
            gather(b, b)

        @pl.loop(0, CH_PER_W, step=NBUF)
        def _(t):
            for b in range(NBUF):
                gather_wait(b)
                scat(t + b, b)
            for b in range(NBUF):
                @pl.when(t + b + NBUF < CH_PER_W)
                def _(b=b):
                    scat_wait(b)
                    gather(t + b + NBUF, b)

        for b in range(NBUF):
            scat_wait(b)

        plsc.subcore_barrier()
        pltpu.sync_copy(acc.at[pl.ds(sid * RPS, RPS)],
                        out_hbm.at[cid, pl.ds(sid * RPS, RPS)])

    return k


_edge_scatter = {h: _make_edge_scatter(h) for h in (32, 64)}


@functools.partial(
    pl.kernel,
    out_type=jax.ShapeDtypeStruct((NC, NP, DEGW), jnp.float32),
    mesh=_mesh,
    scratch_types=[
        pltpu.VMEM((CH_PER_W, CH), jnp.int32),
        pltpu.VMEM((CH, DEGW), jnp.float32),
        pltpu.VMEM((ZCH, DEGW), jnp.float32),
        pltpu.VMEM_SHARED((NP, DEGW), jnp.float32),
        pltpu.SemaphoreType.DMA,
    ],
    compiler_params=_sc_params,
)
def _deg_kernel(dst_hbm, out_hbm, didx, obuf, zbuf, acc, dsem):
    cid = lax.axis_index("c")
    sid = lax.axis_index("s")
    wid = cid * NS + sid

    _zero_fill(zbuf, DEGW)

    @pl.loop(0, obuf.shape[0])
    def _(r):
        obuf[r, pl.ds(0, 16)] = jnp.ones((16,), jnp.float32)

    @pl.loop(0, RPS // ZCH)
    def _(t):
        pltpu.sync_copy(zbuf, acc.at[pl.ds(sid * RPS + t * ZCH, ZCH)])

    pltpu.sync_copy(dst_hbm.at[pl.ds(wid * CH_PER_W, CH_PER_W)], didx)
    plsc.subcore_barrier()

    @pl.loop(0, CH_PER_W, step=NBUF)
    def _(t):
        for b in range(NBUF):
            pltpu.async_copy(obuf, acc.at[didx.at[t + b]], dsem, add=True)
        for b in range(NBUF):
            pltpu.make_async_copy(obuf, acc.at[didx.at[0]], dsem).wait()

    plsc.subcore_barrier()
    pltpu.sync_copy(acc.at[pl.ds(sid * RPS, RPS)],
                    out_hbm.at[cid, pl.ds(sid * RPS, RPS)])


def _stage0_body(deg_ref, x_ref, w_ref, dinv_ref, y_ref):
    deg = (deg_ref[0, pl.ds(0, N_NODES), 0:1]
           + deg_ref[1, pl.ds(0, N_NODES), 0:1] + 1.0)
    dinv = lax.rsqrt(deg)
    dinv_ref[...] = dinv
    y_ref[...] = dinv * jnp.dot(x_ref[...], w_ref[...],
                                preferred_element_type=jnp.float32)


def _stage_mid_body(acc_ref, y_ref, dinv_ref, b_ref, w_ref, yout_ref):
    dinv = dinv_ref[...]
    acc = acc_ref[0, pl.ds(0, N_NODES), :] + acc_ref[1, pl.ds(0, N_NODES), :]
    h = jax.nn.relu(dinv * (acc + y_ref[...]) + b_ref[...])
    yout_ref[...] = dinv * jnp.dot(h, w_ref[...],
                                   preferred_element_type=jnp.float32)


def _stage3_body(acc_ref, y_ref, dinv_ref, b_ref, out_ref):
    acc = acc_ref[0, pl.ds(0, N_NODES), :] + acc_ref[1, pl.ds(0, N_NODES), :]
    z = dinv_ref[...] * (acc + y_ref[...]) + b_ref[...]
    m = jnp.max(z, axis=1, keepdims=True)
    e = jnp.exp(z - m)
    out_ref[...] = (z - m) - jnp.log(jnp.sum(e, axis=1, keepdims=True))


def _stage0(deg, x, w):
    return pl.pallas_call(
        _stage0_body,
        out_shape=(jax.ShapeDtypeStruct((N_NODES, 1), jnp.float32),
                   jax.ShapeDtypeStruct((N_NODES, w.shape[1]), jnp.float32)),
    )(deg, x, w)


def _stage_mid(acc, y, dinv, b, w):
    return pl.pallas_call(
        _stage_mid_body,
        out_shape=jax.ShapeDtypeStruct((N_NODES, w.shape[1]), jnp.float32),
    )(acc, y, dinv, b.reshape(1, -1), w)


def _stage3(acc, y, dinv, b):
    return pl.pallas_call(
        _stage3_body,
        out_shape=jax.ShapeDtypeStruct((N_NODES, y.shape[1]), jnp.float32),
    )(acc, y, dinv, b.reshape(1, -1))


def kernel(x, edge_index, W1, b1, W2, b2, W3, b3):
    src = edge_index[0].reshape(NCHUNKS, CH)
    dst = edge_index[1].reshape(NCHUNKS, CH)

    degp = _deg_kernel(dst)
    dinv, y1 = _stage0(degp, x, W1)

    acc1 = _edge_scatter[32](src, dst, y1)
    y2 = _stage_mid(acc1, y1, dinv, b1, W2)

    acc2 = _edge_scatter[64](src, dst, y2)
    y3 = _stage_mid(acc2, y2, dinv, b2, W3)

    acc3 = _edge_scatter[64](src, dst, y3)
    return _stage3(acc3, y3, dinv, b3)
